# Initial kernel scaffold; baseline (speedup 1.0000x reference)
import jax
import jax.numpy as jnp
from jax.experimental import pallas as pl


def kernel(x, basis_0, basis_1, basis_2, basis_3, basis_4, basis_5):
    raise NotImplementedError("write your pallas kernel here")


# speedup vs baseline: 21.1220x; 21.1220x over previous
import functools

import jax
import jax.numpy as jnp
import numpy as np
from jax import lax
from jax.experimental import pallas as pl
from jax.experimental.pallas import tpu as pltpu
from jax.experimental.pallas import tpu_sc as plsc

_DIMS = [4, 4, 4, 4, 2, 2]
_RESOS = [32, 51, 70, 89, 108, 128]
_NPTS = 1048576
_NTILES = 32
_PER_TILE = _NPTS // _NTILES
_CHUNK = 128
_NCHUNKS = _PER_TILE // _CHUNK
_NGROUPS = _CHUNK // 16

_IDX_SLOT = [(0, 1), (2, 3), (4, 5), (6, 7), (8,), (9,)]
_NSLOTS = 10

def _axis_interp(v, scale, half, rm1, r):
    t = jnp.mod(v + 1.0, scale)
    p = t / half - 1.0
    p = jnp.clip(p, -1.0, 1.0)
    ia = (p + 1.0) * 0.5 * rm1
    ia = jnp.clip(ia, 0.0, rm1)
    a0 = ia.astype(jnp.int32)
    wa = ia - a0.astype(jnp.float32)
    a1 = jnp.minimum(a0 + 1, r - 1)
    return a0, a1, wa


def _xor_perm(vec, lanes, xor_mask):
    _, permuted = plsc.sort_key_val(lanes ^ xor_mask, vec)
    return permuted


def _body(x0_hbm, x1_hbm, x2_hbm,
          t0, t1, t2, t3, t4, t5,
          o0, o1, o2, o3, o4, o5,
          px, py, pz, idxb, wbuf,
          v0, v1, v2, v3, v4, v5,
          b0, b1, b2, b3, b4, b5, sem):
    tables = [t0, t1, t2, t3, t4, t5]
    outs = [o0, o1, o2, o3, o4, o5]
    vbufs = [v0, v1, v2, v3, v4, v5]
    lbufs = [b0, b1, b2, b3, b4, b5]
    wid = lax.axis_index("s") * 2 + lax.axis_index("c")
    tile_base = wid * _PER_TILE
    lanes = lax.iota(jnp.int32, 16)

    def wpat(bits):
        bf = bits.astype(jnp.float32)
        return 1.0 - bf, 2.0 * bf - 1.0

    cx4, sx4 = wpat((lanes >> 2) & 1)
    cy4, sy4 = wpat((lanes >> 3) & 1)
    cx2, sx2 = wpat((lanes >> 1) & 1)
    cy2, sy2 = wpat((lanes >> 2) & 1)
    cz2, sz2 = wpat((lanes >> 3) & 1)

    def chunk_body(ci, carry):
        pbase = tile_base + ci * _CHUNK
        pltpu.sync_copy(x0_hbm.at[pl.ds(pbase, _CHUNK)], px)
        pltpu.sync_copy(x1_hbm.at[pl.ds(pbase, _CHUNK)], py)
        pltpu.sync_copy(x2_hbm.at[pl.ds(pbase, _CHUNK)], pz)

        def compute_group(g, c2):
            sl = pl.ds(g * 16, 16)
            vx = px[sl]
            vy = py[sl]
            vz = pz[sl]
            for l, (cdim, r) in enumerate(zip(_DIMS, _RESOS)):
                scale = float(np.float32(2.0) / np.float32(r))
                half = float(np.float32(scale) / np.float32(2.0))
                rm1 = float(r - 1)
                x0i, _, wx = _axis_interp(vx, scale, half, rm1, r)
                y0i, _, wy = _axis_interp(vy, scale, half, rm1, r)
                z0i, z1i, wz = _axis_interp(vz, scale, half, rm1, r)
                base = z0i * (r * r) + y0i * r + x0i
                slots = _IDX_SLOT[l]
                idxb[pl.ds(slots[0] * _CHUNK + g * 16, 16)] = base
                if cdim == 4:
                    idxb[pl.ds(slots[1] * _CHUNK + g * 16, 16)] = (
                        base + (z1i - z0i) * (r * r))
                wbuf[pl.ds((l * 3 + 0) * _CHUNK + g * 16, 16)] = wx
                wbuf[pl.ds((l * 3 + 1) * _CHUNK + g * 16, 16)] = wy
                wbuf[pl.ds((l * 3 + 2) * _CHUNK + g * 16, 16)] = wz
            return c2

        lax.fori_loop(0, _NGROUPS, compute_group, 0)

        copies = []
        for l, cdim in enumerate(_DIMS):
            for z, slot in enumerate(_IDX_SLOT[l]):
                copies.append(pltpu.async_copy(
                    tables[l].at[idxb.at[pl.ds(slot * _CHUNK, _CHUNK)]],
                    vbufs[l].at[pl.ds(z * _CHUNK, _CHUNK)], sem))
        for cp in copies:
            cp.wait()

        def point_body(p, c2):
            for l, cdim in enumerate(_DIMS):
                wb = l * 3 * _CHUNK + p
                bx = plsc.load_gather(wbuf, [jnp.full((16,), wb, jnp.int32)])
                by = plsc.load_gather(
                    wbuf, [jnp.full((16,), wb + _CHUNK, jnp.int32)])
                bz = plsc.load_gather(
                    wbuf, [jnp.full((16,), wb + 2 * _CHUNK, jnp.int32)])
                if cdim == 4:
                    r0 = vbufs[l][p]
                    r1 = vbufs[l][_CHUNK + p]
                    m = r0 + bz * (r1 - r0)
                    wv = (cx4 + sx4 * bx) * (cy4 + sy4 * by)
                    acc = m * wv
                    acc = acc + _xor_perm(acc, lanes, 4)
                    acc = acc + _xor_perm(acc, lanes, 8)
                    plsc.store_scatter(
                        lbufs[l], [p * 4 + (lanes & 3)], acc,
                        mask=lanes < 4)
                else:
                    rr = vbufs[l][p]
                    wv = ((cx2 + sx2 * bx) * (cy2 + sy2 * by)
                          * (cz2 + sz2 * bz))
                    acc = rr * wv
                    acc = acc + _xor_perm(acc, lanes, 2)
                    acc = acc + _xor_perm(acc, lanes, 4)
                    acc = acc + _xor_perm(acc, lanes, 8)
                    plsc.store_scatter(
                        lbufs[l], [p * 2 + (lanes & 1)], acc,
                        mask=lanes < 2)
            return c2

        lax.fori_loop(0, _CHUNK, point_body, 0, unroll=2)

        for l, cdim in enumerate(_DIMS):
            pltpu.sync_copy(
                lbufs[l], outs[l].at[pl.ds(pbase * cdim, _CHUNK * cdim)])
        return carry

    lax.fori_loop(0, _NCHUNKS, chunk_body, 0)


_mesh = plsc.VectorSubcoreMesh(core_axis_name="c", subcore_axis_name="s")

_encoder = functools.partial(
    pl.kernel,
    mesh=_mesh,
    compiler_params=pltpu.CompilerParams(
        needs_layout_passes=False, use_tc_tiling_on_sc=False),
    out_type=tuple(
        jax.ShapeDtypeStruct((_NPTS * d,), jnp.float32) for d in _DIMS),
    scratch_types=[
        pltpu.VMEM((_CHUNK,), jnp.float32),
        pltpu.VMEM((_CHUNK,), jnp.float32),
        pltpu.VMEM((_CHUNK,), jnp.float32),
        pltpu.VMEM((_NSLOTS * _CHUNK,), jnp.int32),
        pltpu.VMEM((18 * _CHUNK,), jnp.float32),
        pltpu.VMEM((2 * _CHUNK, 16), jnp.float32),
        pltpu.VMEM((2 * _CHUNK, 16), jnp.float32),
        pltpu.VMEM((2 * _CHUNK, 16), jnp.float32),
        pltpu.VMEM((2 * _CHUNK, 16), jnp.float32),
        pltpu.VMEM((_CHUNK, 16), jnp.float32),
        pltpu.VMEM((_CHUNK, 16), jnp.float32),
        pltpu.VMEM((_CHUNK * 4,), jnp.float32),
        pltpu.VMEM((_CHUNK * 4,), jnp.float32),
        pltpu.VMEM((_CHUNK * 4,), jnp.float32),
        pltpu.VMEM((_CHUNK * 4,), jnp.float32),
        pltpu.VMEM((_CHUNK * 2,), jnp.float32),
        pltpu.VMEM((_CHUNK * 2,), jnp.float32),
        pltpu.SemaphoreType.DMA,
    ],
)(_body)


def _shift_x(vol):
    return jnp.concatenate([vol[:, :, :, 1:], vol[:, :, :, -1:]], axis=3)


def _shift_y(vol):
    return jnp.concatenate([vol[:, :, 1:, :], vol[:, :, -1:, :]], axis=2)


def _shift_z(vol):
    return jnp.concatenate([vol[:, 1:, :, :], vol[:, -1:, :, :]], axis=1)


@jax.jit
def kernel(x, basis_0, basis_1, basis_2, basis_3, basis_4, basis_5):
    bases = [basis_0, basis_1, basis_2, basis_3, basis_4, basis_5]
    tables = []
    for vol, (d, r) in zip(bases, zip(_DIMS, _RESOS)):
        if d == 4:
            vx = _shift_x(vol)
            patches = [vol, vx, _shift_y(vol), _shift_y(vx)]
        else:
            vx = _shift_x(vol)
            vyx = _shift_y(vx)
            patches = [vol, vx, _shift_y(vol), vyx,
                       _shift_z(vol), _shift_z(vx),
                       _shift_z(_shift_y(vol)), _shift_z(vyx)]
        t = jnp.stack(patches, axis=0)
        t = t.transpose(2, 3, 4, 0, 1)
        tables.append(t.reshape(r * r * r, 16))
    flat = _encoder(x[:, 0], x[:, 1], x[:, 2], *tables)
    return jnp.concatenate(
        [f.reshape(_NPTS, d) for f, d in zip(flat, _DIMS)], axis=1)


# speedup vs baseline: 28.1897x; 1.3346x over previous
import functools

import jax
import jax.numpy as jnp
import numpy as np
from jax import lax
from jax.experimental import pallas as pl
from jax.experimental.pallas import tpu as pltpu
from jax.experimental.pallas import tpu_sc as plsc

_DIMS = [4, 4, 4, 4, 2, 2]
_RESOS = [32, 51, 70, 89, 108, 128]
_NPTS = 1048576
_NTILES = 32
_PER_TILE = _NPTS // _NTILES
_CHUNK = 128
_NCHUNKS = _PER_TILE // _CHUNK
_NGROUPS = _CHUNK // 16

_IDX_SLOT = [(0, 1), (2, 3), (4, 5), (6, 7), (8,), (9,)]
_NSLOTS = 10

def _axis_interp(v, scale, half, rm1, r):
    t = jnp.mod(v + 1.0, scale)
    p = t / half - 1.0
    p = jnp.clip(p, -1.0, 1.0)
    ia = (p + 1.0) * 0.5 * rm1
    ia = jnp.clip(ia, 0.0, rm1)
    a0 = ia.astype(jnp.int32)
    wa = ia - a0.astype(jnp.float32)
    a1 = jnp.minimum(a0 + 1, r - 1)
    return a0, a1, wa


def _body(x0_hbm, x1_hbm, x2_hbm,
          t0, t1, t2, t3, t4, t5,
          o0, o1, o2, o3, o4, o5,
          px, py, pz, idxb, wbuf,
          v0, v1, v2, v3, v4, v5,
          b0, b1, b2, b3, b4, b5, sem):
    tables = [t0, t1, t2, t3, t4, t5]
    outs = [o0, o1, o2, o3, o4, o5]
    vbufs = [v0, v1, v2, v3, v4, v5]
    lbufs = [b0, b1, b2, b3, b4, b5]
    wid = lax.axis_index("s") * 2 + lax.axis_index("c")
    tile_base = wid * _PER_TILE
    lanes = lax.iota(jnp.int32, 16)
    z16 = lanes * 0

    def chunk_body(ci, carry):
        pbase = tile_base + ci * _CHUNK
        pltpu.sync_copy(x0_hbm.at[pl.ds(pbase, _CHUNK)], px)
        pltpu.sync_copy(x1_hbm.at[pl.ds(pbase, _CHUNK)], py)
        pltpu.sync_copy(x2_hbm.at[pl.ds(pbase, _CHUNK)], pz)

        def compute_group(g, c2):
            sl = pl.ds(g * 16, 16)
            vx = px[sl]
            vy = py[sl]
            vz = pz[sl]
            for l, (cdim, r) in enumerate(zip(_DIMS, _RESOS)):
                scale = float(np.float32(2.0) / np.float32(r))
                half = float(np.float32(scale) / np.float32(2.0))
                rm1 = float(r - 1)
                x0i, _, wx = _axis_interp(vx, scale, half, rm1, r)
                y0i, _, wy = _axis_interp(vy, scale, half, rm1, r)
                z0i, z1i, wz = _axis_interp(vz, scale, half, rm1, r)
                base = z0i * (r * r) + y0i * r + x0i
                slots = _IDX_SLOT[l]
                idxb[pl.ds(slots[0] * _CHUNK + g * 16, 16)] = base
                if cdim == 4:
                    idxb[pl.ds(slots[1] * _CHUNK + g * 16, 16)] = (
                        base + (z1i - z0i) * (r * r))
                wbuf[pl.ds((l * 3 + 0) * _CHUNK + g * 16, 16)] = wx
                wbuf[pl.ds((l * 3 + 1) * _CHUNK + g * 16, 16)] = wy
                wbuf[pl.ds((l * 3 + 2) * _CHUNK + g * 16, 16)] = wz
            return c2

        lax.fori_loop(0, _NGROUPS, compute_group, 0)

        copies = []
        for l, cdim in enumerate(_DIMS):
            for z, slot in enumerate(_IDX_SLOT[l]):
                copies.append(pltpu.async_copy(
                    tables[l].at[idxb.at[pl.ds(slot * _CHUNK, _CHUNK)]],
                    vbufs[l].at[pl.ds(z * _CHUNK, _CHUNK)], sem))
        for cp in copies:
            cp.wait()

        def combine_group(g, c2):
            rows = g * 16 + lanes
            for l, cdim in enumerate(_DIMS):
                wx = wbuf[pl.ds((l * 3 + 0) * _CHUNK + g * 16, 16)]
                wy = wbuf[pl.ds((l * 3 + 1) * _CHUNK + g * 16, 16)]
                wz = wbuf[pl.ds((l * 3 + 2) * _CHUNK + g * 16, 16)]
                cwx = (1.0 - wx, wx)
                cwy = (1.0 - wy, wy)
                if cdim == 4:
                    cw = [cwy[dy] * cwx[dx]
                          for dy in (0, 1) for dx in (0, 1)]
                    for c in range(4):
                        acc0 = None
                        acc1 = None
                        for j in range(4):
                            col = z16 + (j * 4 + c)
                            v0 = plsc.load_gather(vbufs[l], [rows, col])
                            v1 = plsc.load_gather(
                                vbufs[l], [_CHUNK + rows, col])
                            t0 = cw[j] * v0
                            t1 = cw[j] * v1
                            acc0 = t0 if acc0 is None else acc0 + t0
                            acc1 = t1 if acc1 is None else acc1 + t1
                        out = acc0 + wz * (acc1 - acc0)
                        plsc.store_scatter(lbufs[l], [rows * 4 + c], out)
                else:
                    cwz = (1.0 - wz, wz)
                    cw = [cwz[dz] * cwy[dy] * cwx[dx]
                          for dz in (0, 1) for dy in (0, 1) for dx in (0, 1)]
                    for c in range(2):
                        acc = None
                        for j in range(8):
                            col = z16 + (j * 2 + c)
                            v = plsc.load_gather(vbufs[l], [rows, col])
                            t = cw[j] * v
                            acc = t if acc is None else acc + t
                        plsc.store_scatter(lbufs[l], [rows * 2 + c], acc)
            return c2

        lax.fori_loop(0, _NGROUPS, combine_group, 0)

        for l, cdim in enumerate(_DIMS):
            pltpu.sync_copy(
                lbufs[l], outs[l].at[pl.ds(pbase * cdim, _CHUNK * cdim)])
        return carry

    lax.fori_loop(0, _NCHUNKS, chunk_body, 0)


_mesh = plsc.VectorSubcoreMesh(core_axis_name="c", subcore_axis_name="s")

_encoder = functools.partial(
    pl.kernel,
    mesh=_mesh,
    compiler_params=pltpu.CompilerParams(
        needs_layout_passes=False, use_tc_tiling_on_sc=False),
    out_type=tuple(
        jax.ShapeDtypeStruct((_NPTS * d,), jnp.float32) for d in _DIMS),
    scratch_types=[
        pltpu.VMEM((_CHUNK,), jnp.float32),
        pltpu.VMEM((_CHUNK,), jnp.float32),
        pltpu.VMEM((_CHUNK,), jnp.float32),
        pltpu.VMEM((_NSLOTS * _CHUNK,), jnp.int32),
        pltpu.VMEM((18 * _CHUNK,), jnp.float32),
        pltpu.VMEM((2 * _CHUNK, 16), jnp.float32),
        pltpu.VMEM((2 * _CHUNK, 16), jnp.float32),
        pltpu.VMEM((2 * _CHUNK, 16), jnp.float32),
        pltpu.VMEM((2 * _CHUNK, 16), jnp.float32),
        pltpu.VMEM((_CHUNK, 16), jnp.float32),
        pltpu.VMEM((_CHUNK, 16), jnp.float32),
        pltpu.VMEM((_CHUNK * 4,), jnp.float32),
        pltpu.VMEM((_CHUNK * 4,), jnp.float32),
        pltpu.VMEM((_CHUNK * 4,), jnp.float32),
        pltpu.VMEM((_CHUNK * 4,), jnp.float32),
        pltpu.VMEM((_CHUNK * 2,), jnp.float32),
        pltpu.VMEM((_CHUNK * 2,), jnp.float32),
        pltpu.SemaphoreType.DMA,
    ],
)(_body)


def _shift_x(vol):
    return jnp.concatenate([vol[:, :, :, 1:], vol[:, :, :, -1:]], axis=3)


def _shift_y(vol):
    return jnp.concatenate([vol[:, :, 1:, :], vol[:, :, -1:, :]], axis=2)


def _shift_z(vol):
    return jnp.concatenate([vol[:, 1:, :, :], vol[:, -1:, :, :]], axis=1)


@jax.jit
def kernel(x, basis_0, basis_1, basis_2, basis_3, basis_4, basis_5):
    bases = [basis_0, basis_1, basis_2, basis_3, basis_4, basis_5]
    tables = []
    for vol, (d, r) in zip(bases, zip(_DIMS, _RESOS)):
        if d == 4:
            vx = _shift_x(vol)
            patches = [vol, vx, _shift_y(vol), _shift_y(vx)]
        else:
            vx = _shift_x(vol)
            vyx = _shift_y(vx)
            patches = [vol, vx, _shift_y(vol), vyx,
                       _shift_z(vol), _shift_z(vx),
                       _shift_z(_shift_y(vol)), _shift_z(vyx)]
        t = jnp.stack(patches, axis=0)
        t = t.transpose(2, 3, 4, 0, 1)
        tables.append(t.reshape(r * r * r, 16))
    flat = _encoder(x[:, 0], x[:, 1], x[:, 2], *tables)
    return jnp.concatenate(
        [f.reshape(_NPTS, d) for f, d in zip(flat, _DIMS)], axis=1)


# speedup vs baseline: 72.9675x; 2.5884x over previous
import functools

import jax
import jax.numpy as jnp
import numpy as np
from jax import lax
from jax.experimental import pallas as pl
from jax.experimental.pallas import tpu as pltpu
from jax.experimental.pallas import tpu_sc as plsc

_DIMS = [4, 4, 4, 4, 2, 2]
_RESOS = [32, 51, 70, 89, 108, 128]
_NPTS = 1048576
_NTILES = 32
_PER_TILE = _NPTS // _NTILES
_CHUNK = 128
_NCHUNKS = _PER_TILE // _CHUNK
_NGROUPS = _CHUNK // 16
_COL0 = [0, 4, 8, 12, 16, 18]
_NFEAT = sum(_DIMS)

_IDX_SLOT = [(0, 1), (2, 3), (4, 5), (6, 7), (8,), (9,)]
_NSLOTS = 10

_B = 2048
_SLAB = _B + 16
_CAT_PAD = 40960

_LVL_BASE = []
_acc = 0
for _d, _r in zip(_DIMS, _RESOS):
    _LVL_BASE.append(_acc)
    _acc += _d * _r ** 3
_CAT_LEN = _acc + _CAT_PAD

_RPT = [-(-r ** 3 // (_NTILES * 8)) * 8 for r in _RESOS]
_TROWS = [_NTILES * rpt for rpt in _RPT]


def _axis_interp(v, scale, half, rm1, r):
    t = jnp.mod(v + 1.0, scale)
    p = t / half - 1.0
    p = jnp.clip(p, -1.0, 1.0)
    ia = (p + 1.0) * 0.5 * rm1
    ia = jnp.clip(ia, 0.0, rm1)
    a0 = ia.astype(jnp.int32)
    wa = ia - a0.astype(jnp.float32)
    a1 = jnp.minimum(a0 + 1, r - 1)
    return a0, a1, wa


def _prep_body(cat, o0, o1, o2, o3, o4, o5, slabs, obuf, sem):
    outs = [o0, o1, o2, o3, o4, o5]
    wid = lax.axis_index("s") * 2 + lax.axis_index("c")
    lanes = lax.iota(jnp.int32, 16)

    for l, (cdim, r) in enumerate(zip(_DIMS, _RESOS)):
        rpt = _RPT[l]
        nblocks = rpt // _B if rpt % _B == 0 else rpt // _B + 1
        start = wid * rpt
        if cdim == 4:
            c_ln = lanes & 3
            off_ln = (lanes >> 3) * r + ((lanes >> 2) & 1)
            s_ln = c_ln * 4 + (lanes >> 2)
            combos = [(c, dy * r + dx)
                      for c in range(4) for dy in (0, 1) for dx in (0, 1)]
        else:
            c_ln = lanes & 1
            off_ln = ((lanes >> 3) * r * r + ((lanes >> 2) & 1) * r
                      + ((lanes >> 1) & 1))
            s_ln = c_ln * 8 + (lanes >> 1)
        if cdim == 2:
            combos = [(c, dz * r * r + dy * r + dx)
                      for c in range(2) for dz in (0, 1)
                      for dy in (0, 1) for dx in (0, 1)]
        bco_ln = _LVL_BASE[l] + c_ln * (r ** 3) + off_ln
        pat = s_ln * _SLAB + (bco_ln & 7)

        def block_body(b, carry, start=start, combos=combos, l=l,
                       cdim=cdim, r=r, pat=pat):
            v0 = start + b * _B
            copies = []
            for s, (c, off) in enumerate(combos):
                bco = _LVL_BASE[l] + c * (r ** 3) + off
                astart = (bco & ~7) + v0
                copies.append(pltpu.async_copy(
                    cat.at[pl.ds(astart, _SLAB)],
                    slabs.at[pl.ds(s * _SLAB, _SLAB)], sem))
            for cp in copies:
                cp.wait()

            def row_body(v, c2):
                row = plsc.load_gather(slabs, [pat + v])
                obuf[pl.ds(v * 16, 16)] = row
                return c2

            lax.fori_loop(0, _B, row_body, 0, unroll=8)
            pltpu.sync_copy(obuf, outs[l].at[pl.ds(v0 * 16, _B * 16)])
            return carry

        lax.fori_loop(0, nblocks, block_body, 0)


def _main_body(xflat, t0, t1, t2, t3, t4, t5, out,
               pbuf, idxb, wbuf, v0, v1, v2, v3, v4, v5, obuf, sem):
    tables = [t0, t1, t2, t3, t4, t5]
    vbufs = [v0, v1, v2, v3, v4, v5]
    wid = lax.axis_index("s") * 2 + lax.axis_index("c")
    tile_base = wid * _PER_TILE
    lanes = lax.iota(jnp.int32, 16)
    z16 = lanes * 0

    def chunk_body(ci, carry):
        pbase = tile_base + ci * _CHUNK
        pltpu.sync_copy(xflat.at[pl.ds(pbase * 3, _CHUNK * 3)], pbuf)

        def compute_group(g, c2):
            pidx = (g * 16 + lanes) * 3
            vx = plsc.load_gather(pbuf, [pidx])
            vy = plsc.load_gather(pbuf, [pidx + 1])
            vz = plsc.load_gather(pbuf, [pidx + 2])
            for l, (cdim, r) in enumerate(zip(_DIMS, _RESOS)):
                scale = float(np.float32(2.0) / np.float32(r))
                half = float(np.float32(scale) / np.float32(2.0))
                rm1 = float(r - 1)
                x0i, _, wx = _axis_interp(vx, scale, half, rm1, r)
                y0i, _, wy = _axis_interp(vy, scale, half, rm1, r)
                z0i, z1i, wz = _axis_interp(vz, scale, half, rm1, r)
                base = z0i * (r * r) + y0i * r + x0i
                slots = _IDX_SLOT[l]
                idxb[pl.ds(slots[0] * _CHUNK + g * 16, 16)] = base
                if cdim == 4:
                    idxb[pl.ds(slots[1] * _CHUNK + g * 16, 16)] = (
                        base + (z1i - z0i) * (r * r))
                wbuf[pl.ds((l * 3 + 0) * _CHUNK + g * 16, 16)] = wx
                wbuf[pl.ds((l * 3 + 1) * _CHUNK + g * 16, 16)] = wy
                wbuf[pl.ds((l * 3 + 2) * _CHUNK + g * 16, 16)] = wz
            return c2

        lax.fori_loop(0, _NGROUPS, compute_group, 0)

        copies = []
        for l, cdim in enumerate(_DIMS):
            for z, slot in enumerate(_IDX_SLOT[l]):
                copies.append(pltpu.async_copy(
                    tables[l].at[idxb.at[pl.ds(slot * _CHUNK, _CHUNK)]],
                    vbufs[l].at[pl.ds(z * _CHUNK, _CHUNK)], sem))
        for cp in copies:
            cp.wait()

        def combine_group(g, c2):
            rows = g * 16 + lanes
            for l, cdim in enumerate(_DIMS):
                wx = wbuf[pl.ds((l * 3 + 0) * _CHUNK + g * 16, 16)]
                wy = wbuf[pl.ds((l * 3 + 1) * _CHUNK + g * 16, 16)]
                wz = wbuf[pl.ds((l * 3 + 2) * _CHUNK + g * 16, 16)]
                cwx = (1.0 - wx, wx)
                cwy = (1.0 - wy, wy)
                if cdim == 4:
                    cw = [cwy[dy] * cwx[dx]
                          for dy in (0, 1) for dx in (0, 1)]
                    for c in range(4):
                        acc0 = None
                        acc1 = None
                        for j in range(4):
                            col = z16 + (j * 4 + c)
                            va = plsc.load_gather(vbufs[l], [rows, col])
                            vb = plsc.load_gather(
                                vbufs[l], [_CHUNK + rows, col])
                            ta = cw[j] * va
                            tb = cw[j] * vb
                            acc0 = ta if acc0 is None else acc0 + ta
                            acc1 = tb if acc1 is None else acc1 + tb
                        res = acc0 + wz * (acc1 - acc0)
                        plsc.store_scatter(
                            obuf, [rows * _NFEAT + (_COL0[l] + c)], res)
                else:
                    cwz = (1.0 - wz, wz)
                    cw = [cwz[dz] * cwy[dy] * cwx[dx]
                          for dz in (0, 1) for dy in (0, 1)
                          for dx in (0, 1)]
                    for c in range(2):
                        acc = None
                        for j in range(8):
                            col = z16 + (j * 2 + c)
                            v = plsc.load_gather(vbufs[l], [rows, col])
                            t = cw[j] * v
                            acc = t if acc is None else acc + t
                        plsc.store_scatter(
                            obuf, [rows * _NFEAT + (_COL0[l] + c)], acc)
            return c2

        lax.fori_loop(0, _NGROUPS, combine_group, 0)

        pltpu.sync_copy(obuf, out.at[pl.ds(pbase * _NFEAT,
                                           _CHUNK * _NFEAT)])
        return carry

    lax.fori_loop(0, _NCHUNKS, chunk_body, 0)


_mesh = plsc.VectorSubcoreMesh(core_axis_name="c", subcore_axis_name="s")
_cparams = pltpu.CompilerParams(
    needs_layout_passes=False, use_tc_tiling_on_sc=False)

_prep = functools.partial(
    pl.kernel,
    mesh=_mesh,
    compiler_params=_cparams,
    out_type=tuple(
        jax.ShapeDtypeStruct((tr * 16,), jnp.float32) for tr in _TROWS),
    scratch_types=[
        pltpu.VMEM((16 * _SLAB,), jnp.float32),
        pltpu.VMEM((_B * 16,), jnp.float32),
        pltpu.SemaphoreType.DMA,
    ],
)(_prep_body)

_main = functools.partial(
    pl.kernel,
    mesh=_mesh,
    compiler_params=_cparams,
    out_type=jax.ShapeDtypeStruct((_NPTS * _NFEAT,), jnp.float32),
    scratch_types=[
        pltpu.VMEM((_CHUNK * 3,), jnp.float32),
        pltpu.VMEM((_NSLOTS * _CHUNK,), jnp.int32),
        pltpu.VMEM((18 * _CHUNK,), jnp.float32),
        pltpu.VMEM((2 * _CHUNK, 16), jnp.float32),
        pltpu.VMEM((2 * _CHUNK, 16), jnp.float32),
        pltpu.VMEM((2 * _CHUNK, 16), jnp.float32),
        pltpu.VMEM((2 * _CHUNK, 16), jnp.float32),
        pltpu.VMEM((_CHUNK, 16), jnp.float32),
        pltpu.VMEM((_CHUNK, 16), jnp.float32),
        pltpu.VMEM((_CHUNK * _NFEAT,), jnp.float32),
        pltpu.SemaphoreType.DMA,
    ],
)(_main_body)


@jax.jit
def kernel(x, basis_0, basis_1, basis_2, basis_3, basis_4, basis_5):
    bases = [basis_0, basis_1, basis_2, basis_3, basis_4, basis_5]
    cat = jnp.concatenate(
        [b.reshape(-1) for b in bases]
        + [jnp.zeros((_CAT_PAD,), jnp.float32)])
    tables = _prep(cat)
    tabs2d = [t.reshape(tr, 16) for t, tr in zip(tables, _TROWS)]
    out = _main(x.reshape(-1), *tabs2d)
    return out.reshape(_NPTS, _NFEAT)


# speedup vs baseline: 75.2162x; 1.0308x over previous
import functools

import jax
import jax.numpy as jnp
import numpy as np
from jax import lax
from jax.experimental import pallas as pl
from jax.experimental.pallas import tpu as pltpu
from jax.experimental.pallas import tpu_sc as plsc

_DIMS = [4, 4, 4, 4, 2, 2]
_RESOS = [32, 51, 70, 89, 108, 128]
_NPTS = 1048576
_NTILES = 32
_PER_TILE = _NPTS // _NTILES
_CHUNK = 128
_NCHUNKS = _PER_TILE // _CHUNK
_NGROUPS = _CHUNK // 16
_COL0 = [0, 4, 8, 12, 16, 18]
_NFEAT = sum(_DIMS)

_IDX_SLOT = [(0, 1), (2, 3), (4, 5), (6, 7), (8,), (9,)]
_NSLOTS = 10

_B = 2048
_SLAB = _B + 16
_CAT_PAD = 40960

_LVL_BASE = []
_acc = 0
for _d, _r in zip(_DIMS, _RESOS):
    _LVL_BASE.append(_acc)
    _acc += _d * _r ** 3
_CAT_LEN = _acc + _CAT_PAD

_RPT = [-(-r ** 3 // (_NTILES * 8)) * 8 for r in _RESOS]
_TROWS = [_NTILES * rpt for rpt in _RPT]


def _axis_interp(v, scale, half, rm1, r):
    t = jnp.mod(v + 1.0, scale)
    p = t / half - 1.0
    p = jnp.clip(p, -1.0, 1.0)
    ia = (p + 1.0) * 0.5 * rm1
    ia = jnp.clip(ia, 0.0, rm1)
    a0 = ia.astype(jnp.int32)
    wa = ia - a0.astype(jnp.float32)
    a1 = jnp.minimum(a0 + 1, r - 1)
    return a0, a1, wa


def _prep_body(cat, o0, o1, o2, o3, o4, o5, slabs, obuf, sem):
    outs = [o0, o1, o2, o3, o4, o5]
    wid = lax.axis_index("s") * 2 + lax.axis_index("c")
    lanes = lax.iota(jnp.int32, 16)

    for l, (cdim, r) in enumerate(zip(_DIMS, _RESOS)):
        rpt = _RPT[l]
        nblocks = rpt // _B if rpt % _B == 0 else rpt // _B + 1
        start = wid * rpt
        if cdim == 4:
            c_ln = lanes & 3
            off_ln = (lanes >> 3) * r + ((lanes >> 2) & 1)
            s_ln = c_ln * 4 + (lanes >> 2)
            combos = [(c, dy * r + dx)
                      for c in range(4) for dy in (0, 1) for dx in (0, 1)]
        else:
            c_ln = lanes & 1
            off_ln = ((lanes >> 3) * r * r + ((lanes >> 2) & 1) * r
                      + ((lanes >> 1) & 1))
            s_ln = c_ln * 8 + (lanes >> 1)
        if cdim == 2:
            combos = [(c, dz * r * r + dy * r + dx)
                      for c in range(2) for dz in (0, 1)
                      for dy in (0, 1) for dx in (0, 1)]
        bco_ln = _LVL_BASE[l] + c_ln * (r ** 3) + off_ln
        pat = s_ln * _SLAB + (bco_ln & 7)

        def block_body(b, carry, start=start, combos=combos, l=l,
                       cdim=cdim, r=r, pat=pat):
            v0 = start + b * _B
            copies = []
            for s, (c, off) in enumerate(combos):
                bco = _LVL_BASE[l] + c * (r ** 3) + off
                astart = (bco & ~7) + v0
                copies.append(pltpu.async_copy(
                    cat.at[pl.ds(astart, _SLAB)],
                    slabs.at[pl.ds(s * _SLAB, _SLAB)], sem))
            for cp in copies:
                cp.wait()

            def row_body(v, c2):
                row = plsc.load_gather(slabs, [pat + v])
                obuf[v] = row
                return c2

            lax.fori_loop(0, _B, row_body, 0, unroll=8)
            pltpu.sync_copy(obuf, outs[l].at[pl.ds(v0, _B)])
            return carry

        lax.fori_loop(0, nblocks, block_body, 0)


def _main_body(xin, t0, t1, t2, t3, t4, t5, out,
               pbuf, idxb, wbuf, v0, v1, v2, v3, v4, v5, obuf, sem):
    tables = [t0, t1, t2, t3, t4, t5]
    vbufs = [v0, v1, v2, v3, v4, v5]
    wid = lax.axis_index("s") * 2 + lax.axis_index("c")
    tile_base = wid * _PER_TILE
    lanes = lax.iota(jnp.int32, 16)
    z16 = lanes * 0

    def chunk_body(ci, carry):
        pbase = tile_base + ci * _CHUNK
        pltpu.sync_copy(xin.at[pl.ds(pbase, _CHUNK)], pbuf)

        def compute_group(g, c2):
            prow = g * 16 + lanes
            vx = plsc.load_gather(pbuf, [prow, z16])
            vy = plsc.load_gather(pbuf, [prow, z16 + 1])
            vz = plsc.load_gather(pbuf, [prow, z16 + 2])
            for l, (cdim, r) in enumerate(zip(_DIMS, _RESOS)):
                scale = float(np.float32(2.0) / np.float32(r))
                half = float(np.float32(scale) / np.float32(2.0))
                rm1 = float(r - 1)
                x0i, _, wx = _axis_interp(vx, scale, half, rm1, r)
                y0i, _, wy = _axis_interp(vy, scale, half, rm1, r)
                z0i, z1i, wz = _axis_interp(vz, scale, half, rm1, r)
                base = z0i * (r * r) + y0i * r + x0i
                slots = _IDX_SLOT[l]
                idxb[pl.ds(slots[0] * _CHUNK + g * 16, 16)] = base
                if cdim == 4:
                    idxb[pl.ds(slots[1] * _CHUNK + g * 16, 16)] = (
                        base + (z1i - z0i) * (r * r))
                wbuf[pl.ds((l * 3 + 0) * _CHUNK + g * 16, 16)] = wx
                wbuf[pl.ds((l * 3 + 1) * _CHUNK + g * 16, 16)] = wy
                wbuf[pl.ds((l * 3 + 2) * _CHUNK + g * 16, 16)] = wz
            return c2

        lax.fori_loop(0, _NGROUPS, compute_group, 0)

        copies = []
        for l, cdim in enumerate(_DIMS):
            for z, slot in enumerate(_IDX_SLOT[l]):
                copies.append(pltpu.async_copy(
                    tables[l].at[idxb.at[pl.ds(slot * _CHUNK, _CHUNK)]],
                    vbufs[l].at[pl.ds(z * _CHUNK, _CHUNK)], sem))
        for cp in copies:
            cp.wait()

        def combine_group(g, c2):
            rows = g * 16 + lanes
            for l, cdim in enumerate(_DIMS):
                wx = wbuf[pl.ds((l * 3 + 0) * _CHUNK + g * 16, 16)]
                wy = wbuf[pl.ds((l * 3 + 1) * _CHUNK + g * 16, 16)]
                wz = wbuf[pl.ds((l * 3 + 2) * _CHUNK + g * 16, 16)]
                cwx = (1.0 - wx, wx)
                cwy = (1.0 - wy, wy)
                if cdim == 4:
                    cw = [cwy[dy] * cwx[dx]
                          for dy in (0, 1) for dx in (0, 1)]
                    for c in range(4):
                        acc0 = None
                        acc1 = None
                        for j in range(4):
                            col = z16 + (j * 4 + c)
                            va = plsc.load_gather(vbufs[l], [rows, col])
                            vb = plsc.load_gather(
                                vbufs[l], [_CHUNK + rows, col])
                            ta = cw[j] * va
                            tb = cw[j] * vb
                            acc0 = ta if acc0 is None else acc0 + ta
                            acc1 = tb if acc1 is None else acc1 + tb
                        res = acc0 + wz * (acc1 - acc0)
                        plsc.store_scatter(
                            obuf, [rows, z16 + (_COL0[l] + c)], res)
                else:
                    cwz = (1.0 - wz, wz)
                    cw = [cwz[dz] * cwy[dy] * cwx[dx]
                          for dz in (0, 1) for dy in (0, 1)
                          for dx in (0, 1)]
                    for c in range(2):
                        acc = None
                        for j in range(8):
                            col = z16 + (j * 2 + c)
                            v = plsc.load_gather(vbufs[l], [rows, col])
                            t = cw[j] * v
                            acc = t if acc is None else acc + t
                        plsc.store_scatter(
                            obuf, [rows, z16 + (_COL0[l] + c)], acc)
            return c2

        lax.fori_loop(0, _NGROUPS, combine_group, 0)

        pltpu.sync_copy(obuf, out.at[pl.ds(pbase, _CHUNK)])
        return carry

    lax.fori_loop(0, _NCHUNKS, chunk_body, 0)


_mesh = plsc.VectorSubcoreMesh(core_axis_name="c", subcore_axis_name="s")
_cparams = pltpu.CompilerParams(
    needs_layout_passes=False, use_tc_tiling_on_sc=False)

_prep = functools.partial(
    pl.kernel,
    mesh=_mesh,
    compiler_params=_cparams,
    out_type=tuple(
        jax.ShapeDtypeStruct((tr, 16), jnp.float32) for tr in _TROWS),
    scratch_types=[
        pltpu.VMEM((16 * _SLAB,), jnp.float32),
        pltpu.VMEM((_B, 16), jnp.float32),
        pltpu.SemaphoreType.DMA,
    ],
)(_prep_body)

_main = functools.partial(
    pl.kernel,
    mesh=_mesh,
    compiler_params=_cparams,
    out_type=jax.ShapeDtypeStruct((_NPTS, _NFEAT), jnp.float32),
    scratch_types=[
        pltpu.VMEM((_CHUNK, 3), jnp.float32),
        pltpu.VMEM((_NSLOTS * _CHUNK,), jnp.int32),
        pltpu.VMEM((18 * _CHUNK,), jnp.float32),
        pltpu.VMEM((2 * _CHUNK, 16), jnp.float32),
        pltpu.VMEM((2 * _CHUNK, 16), jnp.float32),
        pltpu.VMEM((2 * _CHUNK, 16), jnp.float32),
        pltpu.VMEM((2 * _CHUNK, 16), jnp.float32),
        pltpu.VMEM((_CHUNK, 16), jnp.float32),
        pltpu.VMEM((_CHUNK, 16), jnp.float32),
        pltpu.VMEM((_CHUNK, _NFEAT), jnp.float32),
        pltpu.SemaphoreType.DMA,
    ],
)(_main_body)


@jax.jit
def kernel(x, basis_0, basis_1, basis_2, basis_3, basis_4, basis_5):
    bases = [basis_0, basis_1, basis_2, basis_3, basis_4, basis_5]
    cat = jnp.concatenate(
        [b.reshape(-1) for b in bases]
        + [jnp.zeros((_CAT_PAD,), jnp.float32)])
    tables = _prep(cat)
    return _main(x, *tables)


# speedup vs baseline: 78.0369x; 1.0375x over previous
import functools

import jax
import jax.numpy as jnp
import numpy as np
from jax import lax
from jax.experimental import pallas as pl
from jax.experimental.pallas import tpu as pltpu
from jax.experimental.pallas import tpu_sc as plsc

_DIMS = [4, 4, 4, 4, 2, 2]
_RESOS = [32, 51, 70, 89, 108, 128]
_NPTS = 1048576
_NTILES = 32
_PER_TILE = _NPTS // _NTILES
_CHUNK = 128
_NCHUNKS = _PER_TILE // _CHUNK
_NGROUPS = _CHUNK // 16
_COL0 = [0, 4, 8, 12, 16, 18]
_NFEAT = sum(_DIMS)

_IDX_SLOT = [(0, 1), (2, 3), (4, 5), (6, 7), (8,), (9,)]
_NSLOTS = 10

_B = 2048
_SLAB = _B + 16
_CAT_PAD = 40960

_LVL_BASE = []
_acc = 0
for _d, _r in zip(_DIMS, _RESOS):
    _LVL_BASE.append(_acc)
    _acc += _d * _r ** 3
_CAT_LEN = _acc + _CAT_PAD

_RPT = [-(-r ** 3 // (_NTILES * 8)) * 8 for r in _RESOS]
_TROWS = [_NTILES * rpt for rpt in _RPT]


def _axis_interp(v, scale, half, rm1, r):
    t = jnp.mod(v + 1.0, scale)
    p = t / half - 1.0
    p = jnp.clip(p, -1.0, 1.0)
    ia = (p + 1.0) * 0.5 * rm1
    ia = jnp.clip(ia, 0.0, rm1)
    a0 = ia.astype(jnp.int32)
    wa = ia - a0.astype(jnp.float32)
    a1 = jnp.minimum(a0 + 1, r - 1)
    return a0, a1, wa


def _prep_body(cat, o0, o1, o2, o3, o4, o5, slabs, obuf, sem):
    outs = [o0, o1, o2, o3, o4, o5]
    wid = lax.axis_index("s") * 2 + lax.axis_index("c")
    lanes = lax.iota(jnp.int32, 16)

    for l, (cdim, r) in enumerate(zip(_DIMS, _RESOS)):
        rpt = _RPT[l]
        nblocks = rpt // _B if rpt % _B == 0 else rpt // _B + 1
        start = wid * rpt
        if cdim == 4:
            c_ln = lanes & 3
            off_ln = (lanes >> 3) * r + ((lanes >> 2) & 1)
            s_ln = c_ln * 4 + (lanes >> 2)
            combos = [(c, dy * r + dx)
                      for c in range(4) for dy in (0, 1) for dx in (0, 1)]
        else:
            c_ln = lanes & 1
            off_ln = ((lanes >> 3) * r * r + ((lanes >> 2) & 1) * r
                      + ((lanes >> 1) & 1))
            s_ln = c_ln * 8 + (lanes >> 1)
        if cdim == 2:
            combos = [(c, dz * r * r + dy * r + dx)
                      for c in range(2) for dz in (0, 1)
                      for dy in (0, 1) for dx in (0, 1)]
        bco_ln = _LVL_BASE[l] + c_ln * (r ** 3) + off_ln
        pat = s_ln * _SLAB + (bco_ln & 7)

        def block_body(b, carry, start=start, combos=combos, l=l,
                       cdim=cdim, r=r, pat=pat):
            v0 = start + b * _B
            copies = []
            for s, (c, off) in enumerate(combos):
                bco = _LVL_BASE[l] + c * (r ** 3) + off
                astart = (bco & ~7) + v0
                copies.append(pltpu.async_copy(
                    cat.at[pl.ds(astart, _SLAB)],
                    slabs.at[pl.ds(s * _SLAB, _SLAB)], sem))
            for cp in copies:
                cp.wait()

            def row_body(v, c2):
                row = plsc.load_gather(slabs, [pat + v])
                obuf[v] = row
                return c2

            lax.fori_loop(0, _B, row_body, 0, unroll=8)
            pltpu.sync_copy(obuf, outs[l].at[pl.ds(v0, _B)])
            return carry

        lax.fori_loop(0, nblocks, block_body, 0)


def _main_body(xin, t0, t1, t2, t3, t4, t5, out,
               pbuf, idxb, wbuf, v0, v1, v2, v3, v4, v5, obuf, sem):
    tables = [t0, t1, t2, t3, t4, t5]
    vbufs = [v0, v1, v2, v3, v4, v5]
    wid = lax.axis_index("s") * 2 + lax.axis_index("c")
    tile_base = wid * _PER_TILE
    lanes = lax.iota(jnp.int32, 16)
    z16 = lanes * 0

    def chunk_body(ci, carry):
        pbase = tile_base + ci * _CHUNK
        pltpu.sync_copy(xin.at[pl.ds(pbase, _CHUNK)], pbuf)

        def compute_group(g, c2):
            prow = g * 16 + lanes
            vx = plsc.load_gather(pbuf, [prow, z16])
            vy = plsc.load_gather(pbuf, [prow, z16 + 1])
            vz = plsc.load_gather(pbuf, [prow, z16 + 2])
            for l, (cdim, r) in enumerate(zip(_DIMS, _RESOS)):
                scale = float(np.float32(2.0) / np.float32(r))
                half = float(np.float32(scale) / np.float32(2.0))
                rm1 = float(r - 1)
                x0i, _, wx = _axis_interp(vx, scale, half, rm1, r)
                y0i, _, wy = _axis_interp(vy, scale, half, rm1, r)
                z0i, z1i, wz = _axis_interp(vz, scale, half, rm1, r)
                base = z0i * (r * r) + y0i * r + x0i
                slots = _IDX_SLOT[l]
                idxb[pl.ds(slots[0] * _CHUNK + g * 16, 16)] = base
                if cdim == 4:
                    idxb[pl.ds(slots[1] * _CHUNK + g * 16, 16)] = (
                        base + (z1i - z0i) * (r * r))
                wbuf[pl.ds((l * 3 + 0) * _CHUNK + g * 16, 16)] = wx
                wbuf[pl.ds((l * 3 + 1) * _CHUNK + g * 16, 16)] = wy
                wbuf[pl.ds((l * 3 + 2) * _CHUNK + g * 16, 16)] = wz
            return c2

        half = _CHUNK // 2
        hgroups = _NGROUPS // 2
        sub_copies = []
        for sub in range(2):
            lax.fori_loop(sub * hgroups, (sub + 1) * hgroups,
                          compute_group, 0)
            copies = []
            for l, cdim in enumerate(_DIMS):
                for z, slot in enumerate(_IDX_SLOT[l]):
                    copies.append(pltpu.async_copy(
                        tables[l].at[idxb.at[
                            pl.ds(slot * _CHUNK + sub * half, half)]],
                        vbufs[l].at[
                            pl.ds(z * _CHUNK + sub * half, half)], sem))
            sub_copies.append(copies)

        def combine_group(g, c2):
            rows = g * 16 + lanes
            for l, cdim in enumerate(_DIMS):
                wx = wbuf[pl.ds((l * 3 + 0) * _CHUNK + g * 16, 16)]
                wy = wbuf[pl.ds((l * 3 + 1) * _CHUNK + g * 16, 16)]
                wz = wbuf[pl.ds((l * 3 + 2) * _CHUNK + g * 16, 16)]
                cwx = (1.0 - wx, wx)
                cwy = (1.0 - wy, wy)
                if cdim == 4:
                    cw = [cwy[dy] * cwx[dx]
                          for dy in (0, 1) for dx in (0, 1)]
                    for c in range(4):
                        acc0 = None
                        acc1 = None
                        for j in range(4):
                            col = z16 + (j * 4 + c)
                            va = plsc.load_gather(vbufs[l], [rows, col])
                            vb = plsc.load_gather(
                                vbufs[l], [_CHUNK + rows, col])
                            ta = cw[j] * va
                            tb = cw[j] * vb
                            acc0 = ta if acc0 is None else acc0 + ta
                            acc1 = tb if acc1 is None else acc1 + tb
                        res = acc0 + wz * (acc1 - acc0)
                        plsc.store_scatter(
                            obuf, [rows, z16 + (_COL0[l] + c)], res)
                else:
                    cwz = (1.0 - wz, wz)
                    cw = [cwz[dz] * cwy[dy] * cwx[dx]
                          for dz in (0, 1) for dy in (0, 1)
                          for dx in (0, 1)]
                    for c in range(2):
                        acc = None
                        for j in range(8):
                            col = z16 + (j * 2 + c)
                            v = plsc.load_gather(vbufs[l], [rows, col])
                            t = cw[j] * v
                            acc = t if acc is None else acc + t
                        plsc.store_scatter(
                            obuf, [rows, z16 + (_COL0[l] + c)], acc)
            return c2

        for sub in range(2):
            for cp in sub_copies[sub]:
                cp.wait()
            lax.fori_loop(sub * hgroups, (sub + 1) * hgroups,
                          combine_group, 0)

        pltpu.sync_copy(obuf, out.at[pl.ds(pbase, _CHUNK)])
        return carry

    lax.fori_loop(0, _NCHUNKS, chunk_body, 0)


_mesh = plsc.VectorSubcoreMesh(core_axis_name="c", subcore_axis_name="s")
_cparams = pltpu.CompilerParams(
    needs_layout_passes=False, use_tc_tiling_on_sc=False)

_prep = functools.partial(
    pl.kernel,
    mesh=_mesh,
    compiler_params=_cparams,
    out_type=tuple(
        jax.ShapeDtypeStruct((tr, 16), jnp.float32) for tr in _TROWS),
    scratch_types=[
        pltpu.VMEM((16 * _SLAB,), jnp.float32),
        pltpu.VMEM((_B, 16), jnp.float32),
        pltpu.SemaphoreType.DMA,
    ],
)(_prep_body)

_main = functools.partial(
    pl.kernel,
    mesh=_mesh,
    compiler_params=_cparams,
    out_type=jax.ShapeDtypeStruct((_NPTS, _NFEAT), jnp.float32),
    scratch_types=[
        pltpu.VMEM((_CHUNK, 3), jnp.float32),
        pltpu.VMEM((_NSLOTS * _CHUNK,), jnp.int32),
        pltpu.VMEM((18 * _CHUNK,), jnp.float32),
        pltpu.VMEM((2 * _CHUNK, 16), jnp.float32),
        pltpu.VMEM((2 * _CHUNK, 16), jnp.float32),
        pltpu.VMEM((2 * _CHUNK, 16), jnp.float32),
        pltpu.VMEM((2 * _CHUNK, 16), jnp.float32),
        pltpu.VMEM((_CHUNK, 16), jnp.float32),
        pltpu.VMEM((_CHUNK, 16), jnp.float32),
        pltpu.VMEM((_CHUNK, _NFEAT), jnp.float32),
        pltpu.SemaphoreType.DMA,
    ],
)(_main_body)


@jax.jit
def kernel(x, basis_0, basis_1, basis_2, basis_3, basis_4, basis_5):
    bases = [basis_0, basis_1, basis_2, basis_3, basis_4, basis_5]
    cat = jnp.concatenate(
        [b.reshape(-1) for b in bases]
        + [jnp.zeros((_CAT_PAD,), jnp.float32)])
    tables = _prep(cat)
    return _main(x, *tables)


# speedup vs baseline: 81.9975x; 1.0508x over previous
import functools

import jax
import jax.numpy as jnp
import numpy as np
from jax import lax
from jax.experimental import pallas as pl
from jax.experimental.pallas import tpu as pltpu
from jax.experimental.pallas import tpu_sc as plsc

_DIMS = [4, 4, 4, 4, 2, 2]
_RESOS = [32, 51, 70, 89, 108, 128]
_NPTS = 1048576
_NTILES = 32
_PER_TILE = _NPTS // _NTILES
_CHUNK = 128
_NCHUNKS = _PER_TILE // _CHUNK
_NGROUPS = _CHUNK // 16
_COL0 = [0, 4, 8, 12, 16, 18]
_NFEAT = sum(_DIMS)

_IDX_SLOT = [(0, 1), (2, 3), (4, 5), (6, 7), (8,), (9,)]
_NSLOTS = 10

_B = 2048
_SLAB = _B + 16
_SLABS = 16 * _SLAB
_CAT_PAD = 40960

_LVL_BASE = []
_acc = 0
for _d, _r in zip(_DIMS, _RESOS):
    _LVL_BASE.append(_acc)
    _acc += _d * _r ** 3
_CAT_LEN = _acc + _CAT_PAD

_NBLK = [-(-r ** 3 // (_NTILES * _B)) for r in _RESOS]
_RPT = [nb * _B for nb in _NBLK]
_TROWS = [_NTILES * rpt for rpt in _RPT]


def _axis_interp(v, hr, rm1, r):
    w = (v + 1.0) * hr
    tr = w.astype(jnp.int32).astype(jnp.float32)
    fl = jnp.where(w < tr, tr - 1.0, tr)
    ia = jnp.minimum((w - fl) * rm1, rm1)
    a0 = ia.astype(jnp.int32)
    wa = ia - a0.astype(jnp.float32)
    a1 = jnp.minimum(a0 + 1, r - 1)
    return a0, a1, wa


def _prep_body(cat, o0, o1, o2, o3, o4, o5, slabs, obuf, sem):
    outs = [o0, o1, o2, o3, o4, o5]
    wid = lax.axis_index("s") * 2 + lax.axis_index("c")
    lanes = lax.iota(jnp.int32, 16)

    for l, (cdim, r) in enumerate(zip(_DIMS, _RESOS)):
        rpt = _RPT[l]
        nblocks = _NBLK[l]
        start = wid * rpt
        if cdim == 4:
            c_ln = lanes & 3
            off_ln = (lanes >> 3) * r + ((lanes >> 2) & 1)
            s_ln = c_ln * 4 + (lanes >> 2)
            combos = [(c, dy * r + dx)
                      for c in range(4) for dy in (0, 1) for dx in (0, 1)]
        else:
            c_ln = lanes & 1
            off_ln = ((lanes >> 3) * r * r + ((lanes >> 2) & 1) * r
                      + ((lanes >> 1) & 1))
            s_ln = c_ln * 8 + (lanes >> 1)
        if cdim == 2:
            combos = [(c, dz * r * r + dy * r + dx)
                      for c in range(2) for dz in (0, 1)
                      for dy in (0, 1) for dx in (0, 1)]
        bco_ln = _LVL_BASE[l] + c_ln * (r ** 3) + off_ln
        pat = s_ln * _SLAB + (bco_ln & 7)

        def fire(b, par, combos=combos, l=l):
            for s, (c, off) in enumerate(combos):
                bco = _LVL_BASE[l] + c * (r ** 3) + off
                astart = (bco & ~7) + b * _B
                pltpu.async_copy(
                    cat.at[pl.ds(astart, _SLAB)],
                    slabs.at[pl.ds(par * _SLABS + s * _SLAB, _SLAB)], sem)

        def drain(b, par, combos=combos, l=l):
            for s, (c, off) in enumerate(combos):
                bco = _LVL_BASE[l] + c * (r ** 3) + off
                astart = (bco & ~7) + b * _B
                pltpu.make_async_copy(
                    cat.at[pl.ds(astart, _SLAB)],
                    slabs.at[pl.ds(par * _SLABS + s * _SLAB, _SLAB)],
                    sem).wait()

        fire(start // _B, 0)

        def block_body(b, carry, start=start, l=l, pat=pat,
                       nblocks=nblocks, fire=fire, drain=drain):
            gb = start // _B + b
            par = b & 1

            @pl.when(b < nblocks - 1)
            def _():
                fire(gb + 1, 1 - par)

            drain(gb, par)
            patp = pat + par * _SLABS

            def row_body(v, c2):
                row = plsc.load_gather(slabs, [patp + v])
                obuf[v] = row
                return c2

            lax.fori_loop(0, _B, row_body, 0, unroll=8)
            pltpu.sync_copy(obuf, outs[l].at[pl.ds(gb * _B, _B)])
            return carry

        lax.fori_loop(0, nblocks, block_body, 0)


def _main_body(xin, t0, t1, t2, t3, t4, t5, out,
               pbuf, idxb, wbuf, v0, v1, v2, v3, v4, v5, obuf, sem):
    tables = [t0, t1, t2, t3, t4, t5]
    vbufs = [v0, v1, v2, v3, v4, v5]
    wid = lax.axis_index("s") * 2 + lax.axis_index("c")
    tile_base = wid * _PER_TILE
    lanes = lax.iota(jnp.int32, 16)
    z16 = lanes * 0

    def chunk_body(ci, carry):
        pbase = tile_base + ci * _CHUNK
        pltpu.sync_copy(xin.at[pl.ds(pbase, _CHUNK)], pbuf)

        def compute_group(g, c2):
            prow = g * 16 + lanes
            vx = plsc.load_gather(pbuf, [prow, z16])
            vy = plsc.load_gather(pbuf, [prow, z16 + 1])
            vz = plsc.load_gather(pbuf, [prow, z16 + 2])
            for l, (cdim, r) in enumerate(zip(_DIMS, _RESOS)):
                hr = float(np.float32(r) / np.float32(2.0))
                rm1 = float(r - 1)
                x0i, _, wx = _axis_interp(vx, hr, rm1, r)
                y0i, _, wy = _axis_interp(vy, hr, rm1, r)
                z0i, z1i, wz = _axis_interp(vz, hr, rm1, r)
                base = z0i * (r * r) + y0i * r + x0i
                slots = _IDX_SLOT[l]
                idxb[pl.ds(slots[0] * _CHUNK + g * 16, 16)] = base
                if cdim == 4:
                    idxb[pl.ds(slots[1] * _CHUNK + g * 16, 16)] = (
                        base + (z1i - z0i) * (r * r))
                wbuf[pl.ds((l * 3 + 0) * _CHUNK + g * 16, 16)] = wx
                wbuf[pl.ds((l * 3 + 1) * _CHUNK + g * 16, 16)] = wy
                wbuf[pl.ds((l * 3 + 2) * _CHUNK + g * 16, 16)] = wz
            return c2

        half = _CHUNK // 2
        hgroups = _NGROUPS // 2
        sub_copies = []
        for sub in range(2):
            lax.fori_loop(sub * hgroups, (sub + 1) * hgroups,
                          compute_group, 0)
            copies = []
            for l, cdim in enumerate(_DIMS):
                for z, slot in enumerate(_IDX_SLOT[l]):
                    copies.append(pltpu.async_copy(
                        tables[l].at[idxb.at[
                            pl.ds(slot * _CHUNK + sub * half, half)]],
                        vbufs[l].at[
                            pl.ds(z * _CHUNK + sub * half, half)], sem))
            sub_copies.append(copies)

        def combine_group(g, c2):
            rows = g * 16 + lanes
            for l, cdim in enumerate(_DIMS):
                wx = wbuf[pl.ds((l * 3 + 0) * _CHUNK + g * 16, 16)]
                wy = wbuf[pl.ds((l * 3 + 1) * _CHUNK + g * 16, 16)]
                wz = wbuf[pl.ds((l * 3 + 2) * _CHUNK + g * 16, 16)]
                cwx = (1.0 - wx, wx)
                cwy = (1.0 - wy, wy)
                if cdim == 4:
                    cw = [cwy[dy] * cwx[dx]
                          for dy in (0, 1) for dx in (0, 1)]
                    for c in range(4):
                        acc0 = None
                        acc1 = None
                        for j in range(4):
                            col = z16 + (j * 4 + c)
                            va = plsc.load_gather(vbufs[l], [rows, col])
                            vb = plsc.load_gather(
                                vbufs[l], [_CHUNK + rows, col])
                            ta = cw[j] * va
                            tb = cw[j] * vb
                            acc0 = ta if acc0 is None else acc0 + ta
                            acc1 = tb if acc1 is None else acc1 + tb
                        res = acc0 + wz * (acc1 - acc0)
                        plsc.store_scatter(
                            obuf, [rows, z16 + (_COL0[l] + c)], res)
                else:
                    cwz = (1.0 - wz, wz)
                    cw = [cwz[dz] * cwy[dy] * cwx[dx]
                          for dz in (0, 1) for dy in (0, 1)
                          for dx in (0, 1)]
                    for c in range(2):
                        acc = None
                        for j in range(8):
                            col = z16 + (j * 2 + c)
                            v = plsc.load_gather(vbufs[l], [rows, col])
                            t = cw[j] * v
                            acc = t if acc is None else acc + t
                        plsc.store_scatter(
                            obuf, [rows, z16 + (_COL0[l] + c)], acc)
            return c2

        for sub in range(2):
            for cp in sub_copies[sub]:
                cp.wait()
            lax.fori_loop(sub * hgroups, (sub + 1) * hgroups,
                          combine_group, 0)

        pltpu.sync_copy(obuf, out.at[pl.ds(pbase, _CHUNK)])
        return carry

    lax.fori_loop(0, _NCHUNKS, chunk_body, 0)


_mesh = plsc.VectorSubcoreMesh(core_axis_name="c", subcore_axis_name="s")
_cparams = pltpu.CompilerParams(
    needs_layout_passes=False, use_tc_tiling_on_sc=False)

_prep = functools.partial(
    pl.kernel,
    mesh=_mesh,
    compiler_params=_cparams,
    out_type=tuple(
        jax.ShapeDtypeStruct((tr, 16), jnp.float32) for tr in _TROWS),
    scratch_types=[
        pltpu.VMEM((2 * _SLABS,), jnp.float32),
        pltpu.VMEM((_B, 16), jnp.float32),
        pltpu.SemaphoreType.DMA,
    ],
)(_prep_body)

_main = functools.partial(
    pl.kernel,
    mesh=_mesh,
    compiler_params=_cparams,
    out_type=jax.ShapeDtypeStruct((_NPTS, _NFEAT), jnp.float32),
    scratch_types=[
        pltpu.VMEM((_CHUNK, 3), jnp.float32),
        pltpu.VMEM((_NSLOTS * _CHUNK,), jnp.int32),
        pltpu.VMEM((18 * _CHUNK,), jnp.float32),
        pltpu.VMEM((2 * _CHUNK, 16), jnp.float32),
        pltpu.VMEM((2 * _CHUNK, 16), jnp.float32),
        pltpu.VMEM((2 * _CHUNK, 16), jnp.float32),
        pltpu.VMEM((2 * _CHUNK, 16), jnp.float32),
        pltpu.VMEM((_CHUNK, 16), jnp.float32),
        pltpu.VMEM((_CHUNK, 16), jnp.float32),
        pltpu.VMEM((_CHUNK, _NFEAT), jnp.float32),
        pltpu.SemaphoreType.DMA,
    ],
)(_main_body)


@jax.jit
def kernel(x, basis_0, basis_1, basis_2, basis_3, basis_4, basis_5):
    bases = [basis_0, basis_1, basis_2, basis_3, basis_4, basis_5]
    cat = jnp.concatenate(
        [b.reshape(-1) for b in bases]
        + [jnp.zeros((_CAT_PAD,), jnp.float32)])
    tables = _prep(cat)
    return _main(x, *tables)


# speedup vs baseline: 84.7464x; 1.0335x over previous
import functools

import jax
import jax.numpy as jnp
import numpy as np
from jax import lax
from jax.experimental import pallas as pl
from jax.experimental.pallas import tpu as pltpu
from jax.experimental.pallas import tpu_sc as plsc

_DIMS = [4, 4, 4, 4, 2, 2]
_RESOS = [32, 51, 70, 89, 108, 128]
_NPTS = 1048576
_NTILES = 32
_PER_TILE = _NPTS // _NTILES
_CHUNK = 128
_NCHUNKS = _PER_TILE // _CHUNK
_NGROUPS = _CHUNK // 16
_COL0 = [0, 4, 8, 12, 16, 18]
_NFEAT = sum(_DIMS)

_IDX_SLOT = [(0, 1), (2, 3), (4, 5), (6, 7), (8,), (9,)]
_NSLOTS = 10

_B = 2048
_SLAB = _B + 16
_SLABS = 16 * _SLAB
_CAT_PAD = 40960

_LVL_BASE = []
_acc = 0
for _d, _r in zip(_DIMS, _RESOS):
    _LVL_BASE.append(_acc)
    _acc += _d * _r ** 3
_CAT_LEN = _acc + _CAT_PAD

_NBLK = [-(-r ** 3 // (_NTILES * _B)) for r in _RESOS]
_RPT = [nb * _B for nb in _NBLK]
_TROWS = [_NTILES * rpt for rpt in _RPT]


def _axis_interp(v, hr, rm1, r):
    w = (v + 1.0) * hr
    tr = w.astype(jnp.int32).astype(jnp.float32)
    fl = jnp.where(w < tr, tr - 1.0, tr)
    ia = jnp.minimum((w - fl) * rm1, rm1)
    a0 = ia.astype(jnp.int32)
    wa = ia - a0.astype(jnp.float32)
    a1 = jnp.minimum(a0 + 1, r - 1)
    return a0, a1, wa


def _prep_body(cat, o0, o1, o2, o3, o4, o5, slabs, obuf, sem):
    outs = [o0, o1, o2, o3, o4, o5]
    wid = lax.axis_index("s") * 2 + lax.axis_index("c")
    lanes = lax.iota(jnp.int32, 16)

    for l, (cdim, r) in enumerate(zip(_DIMS, _RESOS)):
        rpt = _RPT[l]
        nblocks = _NBLK[l]
        start = wid * rpt
        if cdim == 4:
            c_ln = lanes & 3
            off_ln = (lanes >> 3) * r + ((lanes >> 2) & 1)
            s_ln = c_ln * 4 + (lanes >> 2)
            combos = [(c, dy * r + dx)
                      for c in range(4) for dy in (0, 1) for dx in (0, 1)]
        else:
            c_ln = lanes & 1
            off_ln = ((lanes >> 3) * r * r + ((lanes >> 2) & 1) * r
                      + ((lanes >> 1) & 1))
            s_ln = c_ln * 8 + (lanes >> 1)
        if cdim == 2:
            combos = [(c, dz * r * r + dy * r + dx)
                      for c in range(2) for dz in (0, 1)
                      for dy in (0, 1) for dx in (0, 1)]
        bco_ln = _LVL_BASE[l] + c_ln * (r ** 3) + off_ln
        pat = s_ln * _SLAB + (bco_ln & 7)

        def fire(b, par, combos=combos, l=l):
            for s, (c, off) in enumerate(combos):
                bco = _LVL_BASE[l] + c * (r ** 3) + off
                astart = (bco & ~7) + b * _B
                pltpu.async_copy(
                    cat.at[pl.ds(astart, _SLAB)],
                    slabs.at[pl.ds(par * _SLABS + s * _SLAB, _SLAB)], sem)

        def drain(b, par, combos=combos, l=l):
            for s, (c, off) in enumerate(combos):
                bco = _LVL_BASE[l] + c * (r ** 3) + off
                astart = (bco & ~7) + b * _B
                pltpu.make_async_copy(
                    cat.at[pl.ds(astart, _SLAB)],
                    slabs.at[pl.ds(par * _SLABS + s * _SLAB, _SLAB)],
                    sem).wait()

        fire(start // _B, 0)

        def block_body(b, carry, start=start, l=l, pat=pat,
                       nblocks=nblocks, fire=fire, drain=drain):
            gb = start // _B + b
            par = b & 1

            @pl.when(b < nblocks - 1)
            def _():
                fire(gb + 1, 1 - par)

            drain(gb, par)
            patp = pat + par * _SLABS

            def row_body(v, c2):
                row = plsc.load_gather(slabs, [patp + v])
                obuf[v] = row
                return c2

            lax.fori_loop(0, _B, row_body, 0, unroll=8)
            pltpu.sync_copy(obuf, outs[l].at[pl.ds(gb * _B, _B)])
            return carry

        lax.fori_loop(0, nblocks, block_body, 0)


def _main_body(xin, t0, t1, t2, t3, t4, t5, out,
               pbuf, idxb, wbuf, v0, v1, v2, v3, v4, v5, obuf,
               sem, xsem, osem):
    tables = [t0, t1, t2, t3, t4, t5]
    vbufs = [v0, v1, v2, v3, v4, v5]
    wid = lax.axis_index("s") * 2 + lax.axis_index("c")
    tile_base = wid * _PER_TILE
    lanes = lax.iota(jnp.int32, 16)
    z16 = lanes * 0

    def x_copy(ci, par):
        pbase = tile_base + ci * _CHUNK
        return pltpu.make_async_copy(
            xin.at[pl.ds(pbase, _CHUNK)],
            pbuf.at[pl.ds(par * _CHUNK, _CHUNK)], xsem)

    def o_copy(ci, par):
        pbase = tile_base + ci * _CHUNK
        return pltpu.make_async_copy(
            obuf.at[pl.ds(par * _CHUNK, _CHUNK)],
            out.at[pl.ds(pbase, _CHUNK)], osem)

    x_copy(0, 0).start()
    x_copy(1, 1).start()

    def chunk_body(ci, carry):
        par = ci & 1

        @pl.when(ci >= 2)
        def _():
            o_copy(ci - 2, par).wait()

        x_copy(ci, par).wait()

        def compute_group(g, c2):
            prow = g * 16 + lanes
            vx = plsc.load_gather(pbuf, [par * _CHUNK + prow, z16])
            vy = plsc.load_gather(pbuf, [par * _CHUNK + prow, z16 + 1])
            vz = plsc.load_gather(pbuf, [par * _CHUNK + prow, z16 + 2])
            for l, (cdim, r) in enumerate(zip(_DIMS, _RESOS)):
                hr = float(np.float32(r) / np.float32(2.0))
                rm1 = float(r - 1)
                x0i, _, wx = _axis_interp(vx, hr, rm1, r)
                y0i, _, wy = _axis_interp(vy, hr, rm1, r)
                z0i, z1i, wz = _axis_interp(vz, hr, rm1, r)
                base = z0i * (r * r) + y0i * r + x0i
                slots = _IDX_SLOT[l]
                idxb[pl.ds(slots[0] * _CHUNK + g * 16, 16)] = base
                if cdim == 4:
                    idxb[pl.ds(slots[1] * _CHUNK + g * 16, 16)] = (
                        base + (z1i - z0i) * (r * r))
                wbuf[pl.ds((l * 3 + 0) * _CHUNK + g * 16, 16)] = wx
                wbuf[pl.ds((l * 3 + 1) * _CHUNK + g * 16, 16)] = wy
                wbuf[pl.ds((l * 3 + 2) * _CHUNK + g * 16, 16)] = wz
            return c2

        half = _CHUNK // 2
        hgroups = _NGROUPS // 2
        sub_copies = []
        for sub in range(2):
            lax.fori_loop(sub * hgroups, (sub + 1) * hgroups,
                          compute_group, 0)
            if sub == 1:
                @pl.when(ci < _NCHUNKS - 2)
                def _():
                    x_copy(ci + 2, par).start()
            copies = []
            for l, cdim in enumerate(_DIMS):
                for z, slot in enumerate(_IDX_SLOT[l]):
                    copies.append(pltpu.async_copy(
                        tables[l].at[idxb.at[
                            pl.ds(slot * _CHUNK + sub * half, half)]],
                        vbufs[l].at[
                            pl.ds(z * _CHUNK + sub * half, half)], sem))
            sub_copies.append(copies)

        def combine_group(g, c2):
            rows = g * 16 + lanes
            for l, cdim in enumerate(_DIMS):
                wx = wbuf[pl.ds((l * 3 + 0) * _CHUNK + g * 16, 16)]
                wy = wbuf[pl.ds((l * 3 + 1) * _CHUNK + g * 16, 16)]
                wz = wbuf[pl.ds((l * 3 + 2) * _CHUNK + g * 16, 16)]
                cwx = (1.0 - wx, wx)
                cwy = (1.0 - wy, wy)
                if cdim == 4:
                    cw = [cwy[dy] * cwx[dx]
                          for dy in (0, 1) for dx in (0, 1)]
                    for c in range(4):
                        acc0 = None
                        acc1 = None
                        for j in range(4):
                            col = z16 + (j * 4 + c)
                            va = plsc.load_gather(vbufs[l], [rows, col])
                            vb = plsc.load_gather(
                                vbufs[l], [_CHUNK + rows, col])
                            ta = cw[j] * va
                            tb = cw[j] * vb
                            acc0 = ta if acc0 is None else acc0 + ta
                            acc1 = tb if acc1 is None else acc1 + tb
                        res = acc0 + wz * (acc1 - acc0)
                        plsc.store_scatter(
                            obuf, [par * _CHUNK + rows,
                                   z16 + (_COL0[l] + c)], res)
                else:
                    cwz = (1.0 - wz, wz)
                    cw = [cwz[dz] * cwy[dy] * cwx[dx]
                          for dz in (0, 1) for dy in (0, 1)
                          for dx in (0, 1)]
                    for c in range(2):
                        acc = None
                        for j in range(8):
                            col = z16 + (j * 2 + c)
                            v = plsc.load_gather(vbufs[l], [rows, col])
                            t = cw[j] * v
                            acc = t if acc is None else acc + t
                        plsc.store_scatter(
                            obuf, [par * _CHUNK + rows,
                                   z16 + (_COL0[l] + c)], acc)
            return c2

        for sub in range(2):
            for cp in sub_copies[sub]:
                cp.wait()
            lax.fori_loop(sub * hgroups, (sub + 1) * hgroups,
                          combine_group, 0)

        o_copy(ci, par).start()
        return carry

    lax.fori_loop(0, _NCHUNKS, chunk_body, 0)
    o_copy(_NCHUNKS - 2, 0).wait()
    o_copy(_NCHUNKS - 1, 1).wait()


_mesh = plsc.VectorSubcoreMesh(core_axis_name="c", subcore_axis_name="s")
_cparams = pltpu.CompilerParams(
    needs_layout_passes=False, use_tc_tiling_on_sc=False)

_prep = functools.partial(
    pl.kernel,
    mesh=_mesh,
    compiler_params=_cparams,
    out_type=tuple(
        jax.ShapeDtypeStruct((tr, 16), jnp.float32) for tr in _TROWS),
    scratch_types=[
        pltpu.VMEM((2 * _SLABS,), jnp.float32),
        pltpu.VMEM((_B, 16), jnp.float32),
        pltpu.SemaphoreType.DMA,
    ],
)(_prep_body)

_main = functools.partial(
    pl.kernel,
    mesh=_mesh,
    compiler_params=_cparams,
    out_type=jax.ShapeDtypeStruct((_NPTS, _NFEAT), jnp.float32),
    scratch_types=[
        pltpu.VMEM((2 * _CHUNK, 3), jnp.float32),
        pltpu.VMEM((_NSLOTS * _CHUNK,), jnp.int32),
        pltpu.VMEM((18 * _CHUNK,), jnp.float32),
        pltpu.VMEM((2 * _CHUNK, 16), jnp.float32),
        pltpu.VMEM((2 * _CHUNK, 16), jnp.float32),
        pltpu.VMEM((2 * _CHUNK, 16), jnp.float32),
        pltpu.VMEM((2 * _CHUNK, 16), jnp.float32),
        pltpu.VMEM((_CHUNK, 16), jnp.float32),
        pltpu.VMEM((_CHUNK, 16), jnp.float32),
        pltpu.VMEM((2 * _CHUNK, _NFEAT), jnp.float32),
        pltpu.SemaphoreType.DMA,
        pltpu.SemaphoreType.DMA,
        pltpu.SemaphoreType.DMA,
    ],
)(_main_body)


@jax.jit
def kernel(x, basis_0, basis_1, basis_2, basis_3, basis_4, basis_5):
    bases = [basis_0, basis_1, basis_2, basis_3, basis_4, basis_5]
    cat = jnp.concatenate(
        [b.reshape(-1) for b in bases]
        + [jnp.zeros((_CAT_PAD,), jnp.float32)])
    tables = _prep(cat)
    return _main(x, *tables)


# speedup vs baseline: 86.3639x; 1.0191x over previous
import functools

import jax
import jax.numpy as jnp
import numpy as np
from jax import lax
from jax.experimental import pallas as pl
from jax.experimental.pallas import tpu as pltpu
from jax.experimental.pallas import tpu_sc as plsc

_DIMS = [4, 4, 4, 4, 2, 2]
_RESOS = [32, 51, 70, 89, 108, 128]
_NPTS = 1048576
_NTILES = 32
_PER_TILE = _NPTS // _NTILES
_CHUNK = 256
_NCHUNKS = _PER_TILE // _CHUNK
_NGROUPS = _CHUNK // 16
_COL0 = [0, 4, 8, 12, 16, 18]
_NFEAT = sum(_DIMS)

_IDX_SLOT = [(0, 1), (2, 3), (4, 5), (6, 7), (8,), (9,)]
_NSLOTS = 10

_B = 2048
_SLAB = _B + 16
_SLABS = 16 * _SLAB
_CAT_PAD = 40960

_LVL_BASE = []
_acc = 0
for _d, _r in zip(_DIMS, _RESOS):
    _LVL_BASE.append(_acc)
    _acc += _d * _r ** 3
_CAT_LEN = _acc + _CAT_PAD

_NBLK = [-(-r ** 3 // (_NTILES * _B)) for r in _RESOS]
_RPT = [nb * _B for nb in _NBLK]
_TROWS = [_NTILES * rpt for rpt in _RPT]


def _axis_interp(v, hr, rm1, r):
    w = (v + 1.0) * hr
    tr = w.astype(jnp.int32).astype(jnp.float32)
    fl = jnp.where(w < tr, tr - 1.0, tr)
    ia = jnp.minimum((w - fl) * rm1, rm1)
    a0 = ia.astype(jnp.int32)
    wa = ia - a0.astype(jnp.float32)
    a1 = jnp.minimum(a0 + 1, r - 1)
    return a0, a1, wa


def _prep_body(cat, o0, o1, o2, o3, o4, o5, slabs, obuf, sem):
    outs = [o0, o1, o2, o3, o4, o5]
    wid = lax.axis_index("s") * 2 + lax.axis_index("c")
    lanes = lax.iota(jnp.int32, 16)

    for l, (cdim, r) in enumerate(zip(_DIMS, _RESOS)):
        rpt = _RPT[l]
        nblocks = _NBLK[l]
        start = wid * rpt
        if cdim == 4:
            c_ln = lanes & 3
            off_ln = (lanes >> 3) * r + ((lanes >> 2) & 1)
            s_ln = c_ln * 4 + (lanes >> 2)
            combos = [(c, dy * r + dx)
                      for c in range(4) for dy in (0, 1) for dx in (0, 1)]
        else:
            c_ln = lanes & 1
            off_ln = ((lanes >> 3) * r * r + ((lanes >> 2) & 1) * r
                      + ((lanes >> 1) & 1))
            s_ln = c_ln * 8 + (lanes >> 1)
        if cdim == 2:
            combos = [(c, dz * r * r + dy * r + dx)
                      for c in range(2) for dz in (0, 1)
                      for dy in (0, 1) for dx in (0, 1)]
        bco_ln = _LVL_BASE[l] + c_ln * (r ** 3) + off_ln
        pat = s_ln * _SLAB + (bco_ln & 7)

        def fire(b, par, combos=combos, l=l):
            for s, (c, off) in enumerate(combos):
                bco = _LVL_BASE[l] + c * (r ** 3) + off
                astart = (bco & ~7) + b * _B
                pltpu.async_copy(
                    cat.at[pl.ds(astart, _SLAB)],
                    slabs.at[pl.ds(par * _SLABS + s * _SLAB, _SLAB)], sem)

        def drain(b, par, combos=combos, l=l):
            for s, (c, off) in enumerate(combos):
                bco = _LVL_BASE[l] + c * (r ** 3) + off
                astart = (bco & ~7) + b * _B
                pltpu.make_async_copy(
                    cat.at[pl.ds(astart, _SLAB)],
                    slabs.at[pl.ds(par * _SLABS + s * _SLAB, _SLAB)],
                    sem).wait()

        fire(start // _B, 0)

        def block_body(b, carry, start=start, l=l, pat=pat,
                       nblocks=nblocks, fire=fire, drain=drain):
            gb = start // _B + b
            par = b & 1

            @pl.when(b < nblocks - 1)
            def _():
                fire(gb + 1, 1 - par)

            drain(gb, par)
            patp = pat + par * _SLABS

            def row_body(v, c2):
                row = plsc.load_gather(slabs, [patp + v])
                obuf[v] = row
                return c2

            lax.fori_loop(0, _B, row_body, 0, unroll=8)
            pltpu.sync_copy(obuf, outs[l].at[pl.ds(gb * _B, _B)])
            return carry

        lax.fori_loop(0, nblocks, block_body, 0)


def _main_body(xin, t0, t1, t2, t3, t4, t5, out,
               pbuf, idxb, wbuf, v0, v1, v2, v3, v4, v5, obuf,
               sem, xsem, osem):
    tables = [t0, t1, t2, t3, t4, t5]
    vbufs = [v0, v1, v2, v3, v4, v5]
    wid = lax.axis_index("s") * 2 + lax.axis_index("c")
    tile_base = wid * _PER_TILE
    lanes = lax.iota(jnp.int32, 16)
    z16 = lanes * 0

    def x_copy(ci, par):
        pbase = tile_base + ci * _CHUNK
        return pltpu.make_async_copy(
            xin.at[pl.ds(pbase, _CHUNK)],
            pbuf.at[pl.ds(par * _CHUNK, _CHUNK)], xsem)

    def o_copy(ci, par):
        pbase = tile_base + ci * _CHUNK
        return pltpu.make_async_copy(
            obuf.at[pl.ds(par * _CHUNK, _CHUNK)],
            out.at[pl.ds(pbase, _CHUNK)], osem)

    x_copy(0, 0).start()
    x_copy(1, 1).start()

    def chunk_body(ci, carry):
        par = ci & 1

        @pl.when(ci >= 2)
        def _():
            o_copy(ci - 2, par).wait()

        x_copy(ci, par).wait()

        def compute_group(g, c2):
            prow = g * 16 + lanes
            vx = plsc.load_gather(pbuf, [par * _CHUNK + prow, z16])
            vy = plsc.load_gather(pbuf, [par * _CHUNK + prow, z16 + 1])
            vz = plsc.load_gather(pbuf, [par * _CHUNK + prow, z16 + 2])
            for l, (cdim, r) in enumerate(zip(_DIMS, _RESOS)):
                hr = float(np.float32(r) / np.float32(2.0))
                rm1 = float(r - 1)
                x0i, _, wx = _axis_interp(vx, hr, rm1, r)
                y0i, _, wy = _axis_interp(vy, hr, rm1, r)
                z0i, z1i, wz = _axis_interp(vz, hr, rm1, r)
                base = z0i * (r * r) + y0i * r + x0i
                slots = _IDX_SLOT[l]
                idxb[pl.ds(slots[0] * _CHUNK + g * 16, 16)] = base
                if cdim == 4:
                    idxb[pl.ds(slots[1] * _CHUNK + g * 16, 16)] = (
                        base + (z1i - z0i) * (r * r))
                wbuf[pl.ds((l * 3 + 0) * _CHUNK + g * 16, 16)] = wx
                wbuf[pl.ds((l * 3 + 1) * _CHUNK + g * 16, 16)] = wy
                wbuf[pl.ds((l * 3 + 2) * _CHUNK + g * 16, 16)] = wz
            return c2

        half = _CHUNK // 2
        hgroups = _NGROUPS // 2
        sub_copies = []
        for sub in range(2):
            lax.fori_loop(sub * hgroups, (sub + 1) * hgroups,
                          compute_group, 0)
            if sub == 1:
                @pl.when(ci < _NCHUNKS - 2)
                def _():
                    x_copy(ci + 2, par).start()
            copies = []
            for l, cdim in enumerate(_DIMS):
                for z, slot in enumerate(_IDX_SLOT[l]):
                    copies.append(pltpu.async_copy(
                        tables[l].at[idxb.at[
                            pl.ds(slot * _CHUNK + sub * half, half)]],
                        vbufs[l].at[
                            pl.ds(z * _CHUNK + sub * half, half)], sem))
            sub_copies.append(copies)

        def combine_group(g, c2):
            rows = g * 16 + lanes
            for l, cdim in enumerate(_DIMS):
                wx = wbuf[pl.ds((l * 3 + 0) * _CHUNK + g * 16, 16)]
                wy = wbuf[pl.ds((l * 3 + 1) * _CHUNK + g * 16, 16)]
                wz = wbuf[pl.ds((l * 3 + 2) * _CHUNK + g * 16, 16)]
                cwx = (1.0 - wx, wx)
                cwy = (1.0 - wy, wy)
                if cdim == 4:
                    cw = [cwy[dy] * cwx[dx]
                          for dy in (0, 1) for dx in (0, 1)]
                    for c in range(4):
                        acc0 = None
                        acc1 = None
                        for j in range(4):
                            col = z16 + (j * 4 + c)
                            va = plsc.load_gather(vbufs[l], [rows, col])
                            vb = plsc.load_gather(
                                vbufs[l], [_CHUNK + rows, col])
                            ta = cw[j] * va
                            tb = cw[j] * vb
                            acc0 = ta if acc0 is None else acc0 + ta
                            acc1 = tb if acc1 is None else acc1 + tb
                        res = acc0 + wz * (acc1 - acc0)
                        plsc.store_scatter(
                            obuf, [par * _CHUNK + rows,
                                   z16 + (_COL0[l] + c)], res)
                else:
                    cwz = (1.0 - wz, wz)
                    cw = [cwz[dz] * cwy[dy] * cwx[dx]
                          for dz in (0, 1) for dy in (0, 1)
                          for dx in (0, 1)]
                    for c in range(2):
                        acc = None
                        for j in range(8):
                            col = z16 + (j * 2 + c)
                            v = plsc.load_gather(vbufs[l], [rows, col])
                            t = cw[j] * v
                            acc = t if acc is None else acc + t
                        plsc.store_scatter(
                            obuf, [par * _CHUNK + rows,
                                   z16 + (_COL0[l] + c)], acc)
            return c2

        for sub in range(2):
            for cp in sub_copies[sub]:
                cp.wait()
            lax.fori_loop(sub * hgroups, (sub + 1) * hgroups,
                          combine_group, 0)

        o_copy(ci, par).start()
        return carry

    lax.fori_loop(0, _NCHUNKS, chunk_body, 0)
    o_copy(_NCHUNKS - 2, 0).wait()
    o_copy(_NCHUNKS - 1, 1).wait()


_mesh = plsc.VectorSubcoreMesh(core_axis_name="c", subcore_axis_name="s")
_cparams = pltpu.CompilerParams(
    needs_layout_passes=False, use_tc_tiling_on_sc=False)

_prep = functools.partial(
    pl.kernel,
    mesh=_mesh,
    compiler_params=_cparams,
    out_type=tuple(
        jax.ShapeDtypeStruct((tr, 16), jnp.float32) for tr in _TROWS),
    scratch_types=[
        pltpu.VMEM((2 * _SLABS,), jnp.float32),
        pltpu.VMEM((_B, 16), jnp.float32),
        pltpu.SemaphoreType.DMA,
    ],
)(_prep_body)

_main = functools.partial(
    pl.kernel,
    mesh=_mesh,
    compiler_params=_cparams,
    out_type=jax.ShapeDtypeStruct((_NPTS, _NFEAT), jnp.float32),
    scratch_types=[
        pltpu.VMEM((2 * _CHUNK, 3), jnp.float32),
        pltpu.VMEM((_NSLOTS * _CHUNK,), jnp.int32),
        pltpu.VMEM((18 * _CHUNK,), jnp.float32),
        pltpu.VMEM((2 * _CHUNK, 16), jnp.float32),
        pltpu.VMEM((2 * _CHUNK, 16), jnp.float32),
        pltpu.VMEM((2 * _CHUNK, 16), jnp.float32),
        pltpu.VMEM((2 * _CHUNK, 16), jnp.float32),
        pltpu.VMEM((_CHUNK, 16), jnp.float32),
        pltpu.VMEM((_CHUNK, 16), jnp.float32),
        pltpu.VMEM((2 * _CHUNK, _NFEAT), jnp.float32),
        pltpu.SemaphoreType.DMA,
        pltpu.SemaphoreType.DMA,
        pltpu.SemaphoreType.DMA,
    ],
)(_main_body)


@jax.jit
def kernel(x, basis_0, basis_1, basis_2, basis_3, basis_4, basis_5):
    bases = [basis_0, basis_1, basis_2, basis_3, basis_4, basis_5]
    cat = jnp.concatenate(
        [b.reshape(-1) for b in bases]
        + [jnp.zeros((_CAT_PAD,), jnp.float32)])
    tables = _prep(cat)
    return _main(x, *tables)
